# Initial kernel scaffold; baseline (speedup 1.0000x reference)
import jax
import jax.numpy as jnp
from jax.experimental import pallas as pl


def kernel(batch_rel_coords, pos_embed):
    raise NotImplementedError("write your pallas kernel here")


# speedup vs baseline: 5.5370x; 5.5370x over previous
import functools

import jax
import jax.numpy as jnp
from jax import lax
from jax.experimental import pallas as pl
from jax.experimental.pallas import tpu as pltpu
from jax.experimental.pallas import tpu_sc as plsc

QUAN_SIZE = 0.02
NHEAD = 16
LANES = 16
NC, NS = 2, 16
NW = NC * NS


@functools.lru_cache(maxsize=None)
def _rpe_sc_kernel(max_len, n_points):
    ppw = n_points // NW
    SLAB = 1024
    n_slabs = ppw // SLAB
    g_per_slab = SLAB // LANES
    tbl_words = max_len * NHEAD

    mesh = plsc.VectorSubcoreMesh(core_axis_name="c", subcore_axis_name="s")

    def _sqrt(s):
        bits = lax.bitcast_convert_type(s, jnp.int32)
        y = lax.bitcast_convert_type((bits >> 1) + 0x1FBD1DF6, jnp.float32)
        for _ in range(3):
            y = 0.5 * (y + s / y)
        return y

    @functools.partial(
        pl.kernel,
        mesh=mesh,
        out_type=jax.ShapeDtypeStruct((n_points * NHEAD,), jnp.float32),
        compiler_params=pltpu.CompilerParams(needs_layout_passes=False),
        scratch_types=[
            pltpu.VMEM((tbl_words,), jnp.float32),
            pltpu.VMEM((ppw,), jnp.float32),
            pltpu.VMEM((ppw,), jnp.float32),
            pltpu.VMEM((ppw,), jnp.float32),
            pltpu.VMEM((SLAB * NHEAD,), jnp.float32),
        ],
    )
    def k(x_hbm, y_hbm, z_hbm, table_hbm, out_hbm, tbl_v, x_v, y_v, z_v, out_v):
        wid = lax.axis_index("s") * NC + lax.axis_index("c")
        base = wid * ppw
        pltpu.sync_copy(table_hbm, tbl_v)
        pltpu.sync_copy(x_hbm.at[pl.ds(base, ppw)], x_v)
        pltpu.sync_copy(y_hbm.at[pl.ds(base, ppw)], y_v)
        pltpu.sync_copy(z_hbm.at[pl.ds(base, ppw)], z_v)
        lanes = lax.iota(jnp.int32, LANES)

        for slab in range(n_slabs):
            def group(g, _):
                off = slab * SLAB + g * LANES
                x = x_v[pl.ds(off, LANES)]
                y = y_v[pl.ds(off, LANES)]
                z = z_v[pl.ds(off, LANES)]
                d = _sqrt(x * x + y * y + z * z) / QUAN_SIZE
                i1 = d.astype(jnp.int32)
                i2 = i1 + 1
                w1 = i2.astype(jnp.float32) - d
                w2 = d - i1.astype(jnp.float32)
                b1 = jnp.minimum(i1, max_len - 1) * NHEAD
                b2 = jnp.minimum(i2, max_len - 1) * NHEAD
                ob = g * (LANES * NHEAD) + lanes * NHEAD
                for c in range(NHEAD):
                    e1 = plsc.load_gather(tbl_v, [b1 + c])
                    e2 = plsc.load_gather(tbl_v, [b2 + c])
                    plsc.store_scatter(out_v, [ob + c], e1 * w1 + e2 * w2)
                return 0

            lax.fori_loop(0, g_per_slab, group, 0)
            pltpu.sync_copy(
                out_v,
                out_hbm.at[pl.ds((base + slab * SLAB) * NHEAD, SLAB * NHEAD)],
            )

    return k


def kernel(batch_rel_coords, pos_embed):
    b, p, _ = batch_rel_coords.shape
    n = b * p
    max_len = pos_embed.shape[0]
    coords = batch_rel_coords.reshape(n, 3)
    table = pos_embed.reshape(-1)
    out = _rpe_sc_kernel(max_len, n)(
        coords[:, 0], coords[:, 1], coords[:, 2], table
    )
    return out.reshape(b, p, NHEAD)
